# baseline (speedup 1.0000x reference)
import functools

import jax
import jax.numpy as jnp
from jax import lax
from jax.experimental import pallas as pl
from jax.experimental.pallas import tpu as pltpu
from jax.experimental.pallas import tpu_sc as plsc

NE = 64
DM = 768
DF = 1024
NT = 4096
TK = 2
NA = NT * TK

BM = 128
NB = NA // BM
TMAX = NB + NE - 1

NW = 32


def _sc_gather(x, sorted_tok):
    rows_per_w = NA // NW
    CH = 64
    mesh = plsc.VectorSubcoreMesh(core_axis_name="c", subcore_axis_name="s")

    @functools.partial(
        pl.kernel,
        mesh=mesh,
        out_type=jax.ShapeDtypeStruct((NA + BM, DM), jnp.float32),
        scratch_types=[
            pltpu.VMEM((CH,), jnp.int32),
            pltpu.VMEM((CH, DM), jnp.float32),
            pltpu.SemaphoreType.DMA,
        ],
    )
    def k(x_hbm, idx_hbm, out_hbm, idx_v, rows_v, sem):
        wid = lax.axis_index("s") * 2 + lax.axis_index("c")
        base = wid * rows_per_w

        def chunk(c, carry):
            o = base + c * CH
            pltpu.sync_copy(idx_hbm.at[pl.ds(o, CH)], idx_v)
            pltpu.async_copy(x_hbm.at[idx_v], rows_v, sem).wait()
            pltpu.sync_copy(rows_v, out_hbm.at[pl.ds(o, CH)])
            return carry

        lax.fori_loop(0, rows_per_w // CH, chunk, None)

    return k(x, sorted_tok)


def _sc_combine(y, pos0, pos1):
    tok_per_w = NT // NW
    C = 32
    mesh = plsc.VectorSubcoreMesh(core_axis_name="c", subcore_axis_name="s")

    @functools.partial(
        pl.kernel,
        mesh=mesh,
        out_type=jax.ShapeDtypeStruct((NT, DM), jnp.float32),
        scratch_types=[
            pltpu.VMEM((C,), jnp.int32),
            pltpu.VMEM((C,), jnp.int32),
            pltpu.VMEM((C, DM), jnp.float32),
            pltpu.VMEM((C, DM), jnp.float32),
            pltpu.VMEM((C, DM), jnp.float32),
            pltpu.SemaphoreType.DMA,
            pltpu.SemaphoreType.DMA,
        ],
    )
    def k(y_hbm, p0_hbm, p1_hbm, out_hbm, i0, i1, r0, r1, ov, s0, s1):
        wid = lax.axis_index("s") * 2 + lax.axis_index("c")
        base = wid * tok_per_w

        def chunk(c, carry):
            o = base + c * C
            pltpu.sync_copy(p0_hbm.at[pl.ds(o, C)], i0)
            pltpu.sync_copy(p1_hbm.at[pl.ds(o, C)], i1)
            cp0 = pltpu.async_copy(y_hbm.at[i0], r0, s0)
            cp1 = pltpu.async_copy(y_hbm.at[i1], r1, s1)
            cp0.wait()
            cp1.wait()

            def per_tok(t, carry2):
                def per_vec(v, carry3):
                    sl = pl.ds(v * 16, 16)
                    ov[t, sl] = r0[t, sl] + r1[t, sl]
                    return carry3

                lax.fori_loop(0, DM // 16, per_vec, None)
                return carry2

            lax.fori_loop(0, C, per_tok, None)
            pltpu.sync_copy(ov, out_hbm.at[pl.ds(o, C)])
            return carry

        lax.fori_loop(0, tok_per_w // C, chunk, None)

    return k(y, pos0, pos1)


POUT = (NB + NE + 3) * BM


def _ffn_body(meta_ref, xs_ref, w1_ref, w2_ref, w3_ref, sw_ref, out_ref,
              xbuf, swbuf, obuf, insem, swsem, outsem):
    e = pl.program_id(0)
    start = meta_ref[0, e]
    nch = meta_ref[1, e]
    poff = meta_ref[2, e]

    w1b = w1_ref[0]
    w2b = w2_ref[0]
    w3b = w3_ref[0]

    def in_cp(j):
        slot = lax.rem(j, 2)
        row = pl.multiple_of(start + j * BM, 8)
        return pltpu.make_async_copy(
            xs_ref.at[pl.ds(row, BM)], xbuf.at[slot], insem.at[slot])

    def sw_cp(j):
        slot = lax.rem(j, 2)
        row = pl.multiple_of(start + j * BM, 8)
        return pltpu.make_async_copy(
            sw_ref.at[pl.ds(row, BM)], swbuf.at[slot], swsem.at[slot])

    def out_cp(j):
        slot = lax.rem(j, 2)
        row = pl.multiple_of(poff + j * BM, 8)
        return pltpu.make_async_copy(
            obuf.at[slot], out_ref.at[pl.ds(row, BM)], outsem.at[slot])

    @pl.when(nch > 0)
    def _():
        in_cp(0).start()
        sw_cp(0).start()

    def do_chunk(j, sx):
        @pl.when(j + 1 < nch)
        def _():
            in_cp(j + 1).start()
            sw_cp(j + 1).start()

        in_cp(j).wait()
        sw_cp(j).wait()

        xb = xbuf[sx]
        g = jax.nn.silu(jnp.dot(xb, w1b, preferred_element_type=jnp.float32))
        v = jnp.dot(xb, w2b, preferred_element_type=jnp.float32)
        h = (g * v) * swbuf[sx]
        o = jnp.dot(h, w3b, preferred_element_type=jnp.float32)

        @pl.when(j >= 2)
        def _():
            out_cp(j).wait()

        obuf[sx] = o
        out_cp(j).start()

    def pair(jj, carry):
        j0 = jj * 2
        do_chunk(j0, 0)

        @pl.when(j0 + 1 < nch)
        def _():
            do_chunk(j0 + 1, 1)

        return carry

    lax.fori_loop(0, (nch + 1) // 2, pair, None)

    @pl.when(nch >= 1)
    def _():
        out_cp(nch - 1).wait()

    @pl.when(nch >= 2)
    def _():
        out_cp(nch - 2).wait()


def _tc_ffn(xs, w1, w2, w3, sw, meta):
    grid_spec = pltpu.PrefetchScalarGridSpec(
        num_scalar_prefetch=1,
        grid=(NE,),
        in_specs=[
            pl.BlockSpec(memory_space=pl.ANY),
            pl.BlockSpec((1, DM, DF), lambda i, m: (i, 0, 0)),
            pl.BlockSpec((1, DM, DF), lambda i, m: (i, 0, 0)),
            pl.BlockSpec((1, DF, DM), lambda i, m: (i, 0, 0)),
            pl.BlockSpec(memory_space=pl.ANY),
        ],
        out_specs=pl.BlockSpec(memory_space=pl.ANY),
        scratch_shapes=[
            pltpu.VMEM((2, BM, DM), jnp.float32),
            pltpu.VMEM((2, BM, 1), jnp.float32),
            pltpu.VMEM((2, BM, DM), jnp.float32),
            pltpu.SemaphoreType.DMA((2,)),
            pltpu.SemaphoreType.DMA((2,)),
            pltpu.SemaphoreType.DMA((2,)),
        ],
    )
    return pl.pallas_call(
        _ffn_body,
        grid_spec=grid_spec,
        out_shape=jax.ShapeDtypeStruct((POUT, DM), jnp.float32),
        compiler_params=pltpu.CompilerParams(
            dimension_semantics=("arbitrary",),
        ),
    )(meta, xs, w1, w2, w3, sw)


def kernel(x, expert_indices, expert_weights, w1, w2, w3):
    flat_e = expert_indices.reshape(-1)
    order = jnp.argsort(flat_e, stable=True).astype(jnp.int32)
    sorted_tok = (order // TK).astype(jnp.int32)
    inv = (
        jnp.zeros((NA,), jnp.int32)
        .at[order]
        .set(jnp.arange(NA, dtype=jnp.int32))
    )
    counts = jnp.bincount(flat_e, length=NE).astype(jnp.int32)
    ends = jnp.cumsum(counts)
    starts = (ends - counts).astype(jnp.int32)

    astart = (starts // 8) * 8
    lead = starts - astart
    nch = (counts + lead + BM - 1) // BM
    poffb = (jnp.cumsum(nch) - nch) * BM
    meta = jnp.stack([astart, nch.astype(jnp.int32),
                      poffb.astype(jnp.int32)])

    sw_sorted = expert_weights.reshape(-1)[order]
    sw = jnp.concatenate(
        [sw_sorted, jnp.zeros((BM,), jnp.float32)]).reshape(NA + BM, 1)

    xs = _sc_gather(x, sorted_tok)
    y = _tc_ffn(xs, w1, w2, w3, sw, meta)

    p_pad = inv - astart[flat_e] + poffb[flat_e].astype(jnp.int32)
    pos = p_pad.reshape(NT, TK)
    out = _sc_combine(y, pos[:, 0], pos[:, 1])
    return out


# speedup vs baseline: 1.0004x; 1.0004x over previous
import functools

import jax
import jax.numpy as jnp
from jax import lax
from jax.experimental import pallas as pl
from jax.experimental.pallas import tpu as pltpu
from jax.experimental.pallas import tpu_sc as plsc

NE = 64
DM = 768
DF = 1024
NT = 4096
TK = 2
NA = NT * TK

BM = 128
NB = NA // BM
TMAX = NB + NE - 1

NW = 32


def _sc_gather(x, sorted_tok):
    rows_per_w = NA // NW
    CH = 64
    mesh = plsc.VectorSubcoreMesh(core_axis_name="c", subcore_axis_name="s")

    @functools.partial(
        pl.kernel,
        mesh=mesh,
        out_type=jax.ShapeDtypeStruct((NA + BM, DM), jnp.float32),
        scratch_types=[
            pltpu.VMEM((CH,), jnp.int32),
            pltpu.VMEM((CH, DM), jnp.float32),
            pltpu.SemaphoreType.DMA,
        ],
    )
    def k(x_hbm, idx_hbm, out_hbm, idx_v, rows_v, sem):
        wid = lax.axis_index("s") * 2 + lax.axis_index("c")
        base = wid * rows_per_w

        def chunk(c, carry):
            o = base + c * CH
            pltpu.sync_copy(idx_hbm.at[pl.ds(o, CH)], idx_v)
            pltpu.async_copy(x_hbm.at[idx_v], rows_v, sem).wait()
            pltpu.sync_copy(rows_v, out_hbm.at[pl.ds(o, CH)])
            return carry

        lax.fori_loop(0, rows_per_w // CH, chunk, None)

    return k(x, sorted_tok)


def _sc_combine(y, pos0, pos1):
    tok_per_w = NT // NW
    C = 32
    mesh = plsc.VectorSubcoreMesh(core_axis_name="c", subcore_axis_name="s")

    @functools.partial(
        pl.kernel,
        mesh=mesh,
        out_type=jax.ShapeDtypeStruct((NT, DM), jnp.float32),
        scratch_types=[
            pltpu.VMEM((C,), jnp.int32),
            pltpu.VMEM((C,), jnp.int32),
            pltpu.VMEM((C, DM), jnp.float32),
            pltpu.VMEM((C, DM), jnp.float32),
            pltpu.VMEM((C, DM), jnp.float32),
            pltpu.SemaphoreType.DMA,
            pltpu.SemaphoreType.DMA,
        ],
    )
    def k(y_hbm, p0_hbm, p1_hbm, out_hbm, i0, i1, r0, r1, ov, s0, s1):
        wid = lax.axis_index("s") * 2 + lax.axis_index("c")
        base = wid * tok_per_w

        def chunk(c, carry):
            o = base + c * C
            pltpu.sync_copy(p0_hbm.at[pl.ds(o, C)], i0)
            pltpu.sync_copy(p1_hbm.at[pl.ds(o, C)], i1)
            cp0 = pltpu.async_copy(y_hbm.at[i0], r0, s0)
            cp1 = pltpu.async_copy(y_hbm.at[i1], r1, s1)
            cp0.wait()
            cp1.wait()

            def per_tok(t, carry2):
                def per_vec(v, carry3):
                    sl = pl.ds(v * 16, 16)
                    ov[t, sl] = r0[t, sl] + r1[t, sl]
                    return carry3

                lax.fori_loop(0, DM // 16, per_vec, None)
                return carry2

            lax.fori_loop(0, C, per_tok, None)
            pltpu.sync_copy(ov, out_hbm.at[pl.ds(o, C)])
            return carry

        lax.fori_loop(0, tok_per_w // C, chunk, None)

    return k(y, pos0, pos1)


POUT = (NB + NE + 3) * BM


def _ffn_body(meta_ref, xs_ref, w1_ref, w2_ref, w3_ref, sw_ref, out_ref,
              xbuf, swbuf, obuf, wbuf1, wbuf2, wbuf3,
              insem, swsem, outsem, wsem):
    e = pl.program_id(0)
    start = meta_ref[0, e]
    nch = meta_ref[1, e]
    poff = meta_ref[2, e]

    def w_cps(ee, slot):
        return (
            pltpu.make_async_copy(w1_ref.at[ee], wbuf1.at[slot], wsem.at[slot]),
            pltpu.make_async_copy(w2_ref.at[ee], wbuf2.at[slot], wsem.at[slot]),
            pltpu.make_async_copy(w3_ref.at[ee], wbuf3.at[slot], wsem.at[slot]),
        )

    wslot = lax.rem(e, 2)

    @pl.when(e == 0)
    def _():
        for cp in w_cps(0, 0):
            cp.start()

    @pl.when(e + 1 < NE)
    def _():
        for cp in w_cps(e + 1, 1 - wslot):
            cp.start()

    for cp in w_cps(e, wslot):
        cp.wait()

    w1b = wbuf1[wslot]
    w2b = wbuf2[wslot]
    w3b = wbuf3[wslot]

    def in_cp(j):
        slot = lax.rem(j, 2)
        row = pl.multiple_of(start + j * BM, 8)
        return pltpu.make_async_copy(
            xs_ref.at[pl.ds(row, BM)], xbuf.at[slot], insem.at[slot])

    def sw_cp(j):
        slot = lax.rem(j, 2)
        row = pl.multiple_of(start + j * BM, 8)
        return pltpu.make_async_copy(
            sw_ref.at[pl.ds(row, BM)], swbuf.at[slot], swsem.at[slot])

    def out_cp(j):
        slot = lax.rem(j, 2)
        row = pl.multiple_of(poff + j * BM, 8)
        return pltpu.make_async_copy(
            obuf.at[slot], out_ref.at[pl.ds(row, BM)], outsem.at[slot])

    @pl.when(nch > 0)
    def _():
        in_cp(0).start()
        sw_cp(0).start()

    def do_chunk(j, sx):
        @pl.when(j + 1 < nch)
        def _():
            in_cp(j + 1).start()
            sw_cp(j + 1).start()

        in_cp(j).wait()
        sw_cp(j).wait()

        xb = xbuf[sx]
        g = jax.nn.silu(jnp.dot(xb, w1b, preferred_element_type=jnp.float32))
        v = jnp.dot(xb, w2b, preferred_element_type=jnp.float32)
        h = (g * v) * swbuf[sx]
        o = jnp.dot(h, w3b, preferred_element_type=jnp.float32)

        @pl.when(j >= 2)
        def _():
            out_cp(j).wait()

        obuf[sx] = o
        out_cp(j).start()

    def pair(jj, carry):
        j0 = jj * 2
        do_chunk(j0, 0)

        @pl.when(j0 + 1 < nch)
        def _():
            do_chunk(j0 + 1, 1)

        return carry

    lax.fori_loop(0, (nch + 1) // 2, pair, None)

    @pl.when(nch >= 1)
    def _():
        out_cp(nch - 1).wait()

    @pl.when(nch >= 2)
    def _():
        out_cp(nch - 2).wait()


def _tc_ffn(xs, w1, w2, w3, sw, meta):
    grid_spec = pltpu.PrefetchScalarGridSpec(
        num_scalar_prefetch=1,
        grid=(NE,),
        in_specs=[
            pl.BlockSpec(memory_space=pl.ANY),
            pl.BlockSpec(memory_space=pl.ANY),
            pl.BlockSpec(memory_space=pl.ANY),
            pl.BlockSpec(memory_space=pl.ANY),
            pl.BlockSpec(memory_space=pl.ANY),
        ],
        out_specs=pl.BlockSpec(memory_space=pl.ANY),
        scratch_shapes=[
            pltpu.VMEM((2, BM, DM), jnp.float32),
            pltpu.VMEM((2, BM, 1), jnp.float32),
            pltpu.VMEM((2, BM, DM), jnp.float32),
            pltpu.VMEM((2, DM, DF), jnp.float32),
            pltpu.VMEM((2, DM, DF), jnp.float32),
            pltpu.VMEM((2, DF, DM), jnp.float32),
            pltpu.SemaphoreType.DMA((2,)),
            pltpu.SemaphoreType.DMA((2,)),
            pltpu.SemaphoreType.DMA((2,)),
            pltpu.SemaphoreType.DMA((2,)),
        ],
    )
    return pl.pallas_call(
        _ffn_body,
        grid_spec=grid_spec,
        out_shape=jax.ShapeDtypeStruct((POUT, DM), jnp.float32),
        compiler_params=pltpu.CompilerParams(
            dimension_semantics=("arbitrary",),
        ),
    )(meta, xs, w1, w2, w3, sw)


def kernel(x, expert_indices, expert_weights, w1, w2, w3):
    flat_e = expert_indices.reshape(-1)
    order = jnp.argsort(flat_e, stable=True).astype(jnp.int32)
    sorted_tok = (order // TK).astype(jnp.int32)
    inv = (
        jnp.zeros((NA,), jnp.int32)
        .at[order]
        .set(jnp.arange(NA, dtype=jnp.int32))
    )
    counts = jnp.bincount(flat_e, length=NE).astype(jnp.int32)
    ends = jnp.cumsum(counts)
    starts = (ends - counts).astype(jnp.int32)

    astart = (starts // 8) * 8
    lead = starts - astart
    nch = (counts + lead + BM - 1) // BM
    poffb = (jnp.cumsum(nch) - nch) * BM
    meta = jnp.stack([astart, nch.astype(jnp.int32),
                      poffb.astype(jnp.int32)])

    sw_sorted = expert_weights.reshape(-1)[order]
    sw = jnp.concatenate(
        [sw_sorted, jnp.zeros((BM,), jnp.float32)]).reshape(NA + BM, 1)

    xs = _sc_gather(x, sorted_tok)
    y = _tc_ffn(xs, w1, w2, w3, sw, meta)

    p_pad = inv - astart[flat_e] + poffb[flat_e].astype(jnp.int32)
    pos = p_pad.reshape(NT, TK)
    out = _sc_combine(y, pos[:, 0], pos[:, 1])
    return out


# speedup vs baseline: 1.0010x; 1.0006x over previous
import functools

import jax
import jax.numpy as jnp
from jax import lax
from jax.experimental import pallas as pl
from jax.experimental.pallas import tpu as pltpu
from jax.experimental.pallas import tpu_sc as plsc

NE = 64
DM = 768
DF = 1024
NT = 4096
TK = 2
NA = NT * TK

BM = 128
NB = NA // BM
TMAX = NB + NE - 1

NW = 32


def _sc_gather(x, sorted_tok):
    rows_per_w = NA // NW
    CH = 64
    mesh = plsc.VectorSubcoreMesh(core_axis_name="c", subcore_axis_name="s")

    @functools.partial(
        pl.kernel,
        mesh=mesh,
        out_type=jax.ShapeDtypeStruct((NA + BM, DM), jnp.float32),
        scratch_types=[
            pltpu.VMEM((CH,), jnp.int32),
            pltpu.VMEM((CH, DM), jnp.float32),
            pltpu.SemaphoreType.DMA,
        ],
    )
    def k(x_hbm, idx_hbm, out_hbm, idx_v, rows_v, sem):
        wid = lax.axis_index("s") * 2 + lax.axis_index("c")
        base = wid * rows_per_w

        def chunk(c, carry):
            o = base + c * CH
            pltpu.sync_copy(idx_hbm.at[pl.ds(o, CH)], idx_v)
            pltpu.async_copy(x_hbm.at[idx_v], rows_v, sem).wait()
            pltpu.sync_copy(rows_v, out_hbm.at[pl.ds(o, CH)])
            return carry

        lax.fori_loop(0, rows_per_w // CH, chunk, None)

    return k(x, sorted_tok)


def _sc_combine(y, pos0, pos1):
    tok_per_w = NT // NW
    C = 32
    mesh = plsc.VectorSubcoreMesh(core_axis_name="c", subcore_axis_name="s")

    @functools.partial(
        pl.kernel,
        mesh=mesh,
        out_type=jax.ShapeDtypeStruct((NT, DM), jnp.float32),
        scratch_types=[
            pltpu.VMEM((C,), jnp.int32),
            pltpu.VMEM((C,), jnp.int32),
            pltpu.VMEM((C, DM), jnp.float32),
            pltpu.VMEM((C, DM), jnp.float32),
            pltpu.VMEM((C, DM), jnp.float32),
            pltpu.SemaphoreType.DMA,
            pltpu.SemaphoreType.DMA,
        ],
    )
    def k(y_hbm, p0_hbm, p1_hbm, out_hbm, i0, i1, r0, r1, ov, s0, s1):
        wid = lax.axis_index("s") * 2 + lax.axis_index("c")
        base = wid * tok_per_w

        def chunk(c, carry):
            o = base + c * C
            pltpu.sync_copy(p0_hbm.at[pl.ds(o, C)], i0)
            pltpu.sync_copy(p1_hbm.at[pl.ds(o, C)], i1)
            cp0 = pltpu.async_copy(y_hbm.at[i0], r0, s0)
            cp1 = pltpu.async_copy(y_hbm.at[i1], r1, s1)
            cp0.wait()
            cp1.wait()

            def per_tok(t, carry2):
                def per_vec(v, carry3):
                    sl = pl.ds(v * 16, 16)
                    ov[t, sl] = r0[t, sl] + r1[t, sl]
                    return carry3

                lax.fori_loop(0, DM // 16, per_vec, None)
                return carry2

            lax.fori_loop(0, C, per_tok, None)
            pltpu.sync_copy(ov, out_hbm.at[pl.ds(o, C)])
            return carry

        lax.fori_loop(0, tok_per_w // C, chunk, None)

    return k(y, pos0, pos1)


POUT = (NB + NE + 3) * BM


def _ffn_body(meta_ref, xs_ref, w1_ref, w2_ref, w3_ref, sw_ref, out_ref,
              xbuf, swbuf, obuf, wbuf1, wbuf2, wbuf3,
              insem, swsem, outsem, wsem):
    e = pl.program_id(0)
    start = meta_ref[0, e]
    nch = meta_ref[1, e]
    poff = meta_ref[2, e]

    def w_cps(ee, slot):
        return (
            pltpu.make_async_copy(w1_ref.at[ee], wbuf1.at[slot], wsem.at[slot]),
            pltpu.make_async_copy(w2_ref.at[ee], wbuf2.at[slot], wsem.at[slot]),
            pltpu.make_async_copy(w3_ref.at[ee], wbuf3.at[slot], wsem.at[slot]),
        )

    wslot = lax.rem(e, 2)

    @pl.when(e == 0)
    def _():
        for cp in w_cps(0, 0):
            cp.start()

    @pl.when(e + 1 < NE)
    def _():
        for cp in w_cps(e + 1, 1 - wslot):
            cp.start()

    for cp in w_cps(e, wslot):
        cp.wait()

    w1b = wbuf1[wslot].astype(jnp.bfloat16)
    w2b = wbuf2[wslot].astype(jnp.bfloat16)
    w3b = wbuf3[wslot].astype(jnp.bfloat16)

    def in_cp(j):
        slot = lax.rem(j, 2)
        row = pl.multiple_of(start + j * BM, 8)
        return pltpu.make_async_copy(
            xs_ref.at[pl.ds(row, BM)], xbuf.at[slot], insem.at[slot])

    def sw_cp(j):
        slot = lax.rem(j, 2)
        row = pl.multiple_of(start + j * BM, 8)
        return pltpu.make_async_copy(
            sw_ref.at[pl.ds(row, BM)], swbuf.at[slot], swsem.at[slot])

    def out_cp(j):
        slot = lax.rem(j, 2)
        row = pl.multiple_of(poff + j * BM, 8)
        return pltpu.make_async_copy(
            obuf.at[slot], out_ref.at[pl.ds(row, BM)], outsem.at[slot])

    @pl.when(nch > 0)
    def _():
        in_cp(0).start()
        sw_cp(0).start()

    def do_chunk(j, sx):
        @pl.when(j + 1 < nch)
        def _():
            in_cp(j + 1).start()
            sw_cp(j + 1).start()

        in_cp(j).wait()
        sw_cp(j).wait()

        xb = xbuf[sx].astype(jnp.bfloat16)
        g = jax.nn.silu(jnp.dot(xb, w1b, preferred_element_type=jnp.float32))
        v = jnp.dot(xb, w2b, preferred_element_type=jnp.float32)
        h = ((g * v) * swbuf[sx]).astype(jnp.bfloat16)
        o = jnp.dot(h, w3b, preferred_element_type=jnp.float32)

        @pl.when(j >= 2)
        def _():
            out_cp(j).wait()

        obuf[sx] = o
        out_cp(j).start()

    def pair(jj, carry):
        j0 = jj * 2
        do_chunk(j0, 0)

        @pl.when(j0 + 1 < nch)
        def _():
            do_chunk(j0 + 1, 1)

        return carry

    lax.fori_loop(0, (nch + 1) // 2, pair, None)

    @pl.when(nch >= 1)
    def _():
        out_cp(nch - 1).wait()

    @pl.when(nch >= 2)
    def _():
        out_cp(nch - 2).wait()


def _tc_ffn(xs, w1, w2, w3, sw, meta):
    grid_spec = pltpu.PrefetchScalarGridSpec(
        num_scalar_prefetch=1,
        grid=(NE,),
        in_specs=[
            pl.BlockSpec(memory_space=pl.ANY),
            pl.BlockSpec(memory_space=pl.ANY),
            pl.BlockSpec(memory_space=pl.ANY),
            pl.BlockSpec(memory_space=pl.ANY),
            pl.BlockSpec(memory_space=pl.ANY),
        ],
        out_specs=pl.BlockSpec(memory_space=pl.ANY),
        scratch_shapes=[
            pltpu.VMEM((2, BM, DM), jnp.float32),
            pltpu.VMEM((2, BM, 1), jnp.float32),
            pltpu.VMEM((2, BM, DM), jnp.float32),
            pltpu.VMEM((2, DM, DF), jnp.float32),
            pltpu.VMEM((2, DM, DF), jnp.float32),
            pltpu.VMEM((2, DF, DM), jnp.float32),
            pltpu.SemaphoreType.DMA((2,)),
            pltpu.SemaphoreType.DMA((2,)),
            pltpu.SemaphoreType.DMA((2,)),
            pltpu.SemaphoreType.DMA((2,)),
        ],
    )
    return pl.pallas_call(
        _ffn_body,
        grid_spec=grid_spec,
        out_shape=jax.ShapeDtypeStruct((POUT, DM), jnp.float32),
        compiler_params=pltpu.CompilerParams(
            dimension_semantics=("arbitrary",),
        ),
    )(meta, xs, w1, w2, w3, sw)


def kernel(x, expert_indices, expert_weights, w1, w2, w3):
    flat_e = expert_indices.reshape(-1)
    order = jnp.argsort(flat_e, stable=True).astype(jnp.int32)
    sorted_tok = (order // TK).astype(jnp.int32)
    inv = (
        jnp.zeros((NA,), jnp.int32)
        .at[order]
        .set(jnp.arange(NA, dtype=jnp.int32))
    )
    counts = jnp.bincount(flat_e, length=NE).astype(jnp.int32)
    ends = jnp.cumsum(counts)
    starts = (ends - counts).astype(jnp.int32)

    astart = (starts // 8) * 8
    lead = starts - astart
    nch = (counts + lead + BM - 1) // BM
    poffb = (jnp.cumsum(nch) - nch) * BM
    meta = jnp.stack([astart, nch.astype(jnp.int32),
                      poffb.astype(jnp.int32)])

    sw_sorted = expert_weights.reshape(-1)[order]
    sw = jnp.concatenate(
        [sw_sorted, jnp.zeros((BM,), jnp.float32)]).reshape(NA + BM, 1)

    xs = _sc_gather(x, sorted_tok)
    y = _tc_ffn(xs, w1, w2, w3, sw, meta)

    p_pad = inv - astart[flat_e] + poffb[flat_e].astype(jnp.int32)
    pos = p_pad.reshape(NT, TK)
    out = _sc_combine(y, pos[:, 0], pos[:, 1])
    return out


# speedup vs baseline: 1.0022x; 1.0012x over previous
import functools

import jax
import jax.numpy as jnp
from jax import lax
from jax.experimental import pallas as pl
from jax.experimental.pallas import tpu as pltpu
from jax.experimental.pallas import tpu_sc as plsc

NE = 64
DM = 768
DF = 1024
NT = 4096
TK = 2
NA = NT * TK

BM = 128
NB = NA // BM
TMAX = NB + NE - 1

NW = 32


def _sc_gather(x, sorted_tok):
    rows_per_w = NA // NW
    CH = 64
    mesh = plsc.VectorSubcoreMesh(core_axis_name="c", subcore_axis_name="s")

    @functools.partial(
        pl.kernel,
        mesh=mesh,
        out_type=jax.ShapeDtypeStruct((NA + BM, DM), jnp.float32),
        scratch_types=[
            pltpu.VMEM((CH,), jnp.int32),
            pltpu.VMEM((CH, DM), jnp.float32),
            pltpu.SemaphoreType.DMA,
        ],
    )
    def k(x_hbm, idx_hbm, out_hbm, idx_v, rows_v, sem):
        wid = lax.axis_index("s") * 2 + lax.axis_index("c")
        base = wid * rows_per_w

        def chunk(c, carry):
            o = base + c * CH
            pltpu.sync_copy(idx_hbm.at[pl.ds(o, CH)], idx_v)
            pltpu.async_copy(x_hbm.at[idx_v], rows_v, sem).wait()
            pltpu.sync_copy(rows_v, out_hbm.at[pl.ds(o, CH)])
            return carry

        lax.fori_loop(0, rows_per_w // CH, chunk, None)

    return k(x, sorted_tok)


def _sc_combine(y, pos0, pos1):
    tok_per_w = NT // NW
    C = 32
    mesh = plsc.VectorSubcoreMesh(core_axis_name="c", subcore_axis_name="s")

    @functools.partial(
        pl.kernel,
        mesh=mesh,
        out_type=jax.ShapeDtypeStruct((NT, DM), jnp.float32),
        scratch_types=[
            pltpu.VMEM((C,), jnp.int32),
            pltpu.VMEM((C,), jnp.int32),
            pltpu.VMEM((C, DM), jnp.float32),
            pltpu.VMEM((C, DM), jnp.float32),
            pltpu.VMEM((C, DM), jnp.float32),
            pltpu.SemaphoreType.DMA,
            pltpu.SemaphoreType.DMA,
        ],
    )
    def k(y_hbm, p0_hbm, p1_hbm, out_hbm, i0, i1, r0, r1, ov, s0, s1):
        wid = lax.axis_index("s") * 2 + lax.axis_index("c")
        base = wid * tok_per_w

        def chunk(c, carry):
            o = base + c * C
            pltpu.sync_copy(p0_hbm.at[pl.ds(o, C)], i0)
            pltpu.sync_copy(p1_hbm.at[pl.ds(o, C)], i1)
            cp0 = pltpu.async_copy(y_hbm.at[i0], r0, s0)
            cp1 = pltpu.async_copy(y_hbm.at[i1], r1, s1)
            cp0.wait()
            cp1.wait()

            def per_tok(t, carry2):
                def per_vec(v, carry3):
                    sl = pl.ds(v * 16, 16)
                    ov[t, sl] = r0[t, sl] + r1[t, sl]
                    return carry3

                lax.fori_loop(0, DM // 16, per_vec, None)
                return carry2

            lax.fori_loop(0, C, per_tok, None)
            pltpu.sync_copy(ov, out_hbm.at[pl.ds(o, C)])
            return carry

        lax.fori_loop(0, tok_per_w // C, chunk, None)

    return k(y, pos0, pos1)


POUT = (NB + NE + 3) * BM


def _ffn_body(meta_ref, xs_ref, w1_ref, w2_ref, w3_ref, sw_ref, out_ref,
              xbuf, swbuf, obuf, wbuf1, wbuf2, wbuf3,
              insem, swsem, outsem, wsem):
    e = pl.program_id(0)
    start = meta_ref[0, e]
    nch = meta_ref[1, e]
    poff = meta_ref[2, e]

    def w_cps(ee, slot):
        return (
            pltpu.make_async_copy(w1_ref.at[ee], wbuf1.at[slot], wsem.at[slot]),
            pltpu.make_async_copy(w2_ref.at[ee], wbuf2.at[slot], wsem.at[slot]),
            pltpu.make_async_copy(w3_ref.at[ee], wbuf3.at[slot], wsem.at[slot]),
        )

    wslot = lax.rem(e, 2)

    @pl.when(e == 0)
    def _():
        for cp in w_cps(0, 0):
            cp.start()

    @pl.when(e + 1 < NE)
    def _():
        for cp in w_cps(e + 1, 1 - wslot):
            cp.start()

    for cp in w_cps(e, wslot):
        cp.wait()

    w1b = wbuf1[wslot].astype(jnp.bfloat16)
    w2b = wbuf2[wslot].astype(jnp.bfloat16)
    w3b = wbuf3[wslot].astype(jnp.bfloat16)

    def in_cp(j):
        slot = lax.rem(j, 2)
        row = pl.multiple_of(start + j * BM, 8)
        return pltpu.make_async_copy(
            xs_ref.at[pl.ds(row, BM)], xbuf.at[slot], insem.at[slot])

    def sw_cp(j):
        slot = lax.rem(j, 2)
        row = pl.multiple_of(start + j * BM, 8)
        return pltpu.make_async_copy(
            sw_ref.at[pl.ds(row, BM)], swbuf.at[slot], swsem.at[slot])

    def out_cp(j):
        slot = lax.rem(j, 2)
        row = pl.multiple_of(poff + j * BM, 8)
        return pltpu.make_async_copy(
            obuf.at[slot], out_ref.at[pl.ds(row, BM)], outsem.at[slot])

    @pl.when(nch > 0)
    def _():
        in_cp(0).start()
        sw_cp(0).start()

    def do_chunk(j, sx):
        @pl.when(j + 1 < nch)
        def _():
            in_cp(j + 1).start()
            sw_cp(j + 1).start()

        in_cp(j).wait()
        sw_cp(j).wait()

        xb = xbuf[sx].astype(jnp.bfloat16)
        g = jax.nn.silu(jnp.dot(xb, w1b, preferred_element_type=jnp.float32))
        v = jnp.dot(xb, w2b, preferred_element_type=jnp.float32)
        h = ((g * v) * swbuf[sx]).astype(jnp.bfloat16)
        o = jnp.dot(h, w3b, preferred_element_type=jnp.float32)

        @pl.when(j >= 2)
        def _():
            out_cp(j).wait()

        obuf[sx] = o
        out_cp(j).start()

    def pair(jj, carry):
        j0 = jj * 2
        do_chunk(j0, 0)

        @pl.when(j0 + 1 < nch)
        def _():
            do_chunk(j0 + 1, 1)

        return carry

    lax.fori_loop(0, (nch + 1) // 2, pair, None)

    @pl.when(nch >= 1)
    def _():
        out_cp(nch - 1).wait()

    @pl.when(nch >= 2)
    def _():
        out_cp(nch - 2).wait()


def _tc_ffn(xs, w1, w2, w3, sw, meta):
    grid_spec = pltpu.PrefetchScalarGridSpec(
        num_scalar_prefetch=1,
        grid=(NE,),
        in_specs=[
            pl.BlockSpec(memory_space=pl.ANY),
            pl.BlockSpec(memory_space=pl.ANY),
            pl.BlockSpec(memory_space=pl.ANY),
            pl.BlockSpec(memory_space=pl.ANY),
            pl.BlockSpec(memory_space=pl.ANY),
        ],
        out_specs=pl.BlockSpec(memory_space=pl.ANY),
        scratch_shapes=[
            pltpu.VMEM((2, BM, DM), jnp.float32),
            pltpu.VMEM((2, BM, 1), jnp.float32),
            pltpu.VMEM((2, BM, DM), jnp.float32),
            pltpu.VMEM((2, DM, DF), jnp.float32),
            pltpu.VMEM((2, DM, DF), jnp.float32),
            pltpu.VMEM((2, DF, DM), jnp.float32),
            pltpu.SemaphoreType.DMA((2,)),
            pltpu.SemaphoreType.DMA((2,)),
            pltpu.SemaphoreType.DMA((2,)),
            pltpu.SemaphoreType.DMA((2,)),
        ],
    )
    return pl.pallas_call(
        _ffn_body,
        grid_spec=grid_spec,
        out_shape=jax.ShapeDtypeStruct((POUT, DM), jnp.float32),
    )(meta, xs, w1, w2, w3, sw)


def kernel(x, expert_indices, expert_weights, w1, w2, w3):
    flat_e = expert_indices.reshape(-1)
    order = jnp.argsort(flat_e, stable=True).astype(jnp.int32)
    sorted_tok = (order // TK).astype(jnp.int32)
    inv = (
        jnp.zeros((NA,), jnp.int32)
        .at[order]
        .set(jnp.arange(NA, dtype=jnp.int32))
    )
    counts = jnp.bincount(flat_e, length=NE).astype(jnp.int32)
    ends = jnp.cumsum(counts)
    starts = (ends - counts).astype(jnp.int32)

    astart = (starts // 8) * 8
    lead = starts - astart
    nch = (counts + lead + BM - 1) // BM
    poffb = (jnp.cumsum(nch) - nch) * BM
    meta = jnp.stack([astart, nch.astype(jnp.int32),
                      poffb.astype(jnp.int32)])

    sw_sorted = expert_weights.reshape(-1)[order]
    sw = jnp.concatenate(
        [sw_sorted, jnp.zeros((BM,), jnp.float32)]).reshape(NA + BM, 1)

    xs = _sc_gather(x, sorted_tok)
    y = _tc_ffn(xs, w1, w2, w3, sw, meta)

    p_pad = inv - astart[flat_e] + poffb[flat_e].astype(jnp.int32)
    pos = p_pad.reshape(NT, TK)
    out = _sc_combine(y, pos[:, 0], pos[:, 1])
    return out


# speedup vs baseline: 1.2820x; 1.2792x over previous
import functools

import jax
import jax.numpy as jnp
from jax import lax
from jax.experimental import pallas as pl
from jax.experimental.pallas import tpu as pltpu
from jax.experimental.pallas import tpu_sc as plsc

NE = 64
DM = 768
DF = 1024
NT = 4096
TK = 2
NA = NT * TK

BM = 128
NB = NA // BM
TMAX = NB + NE - 1

NW = 32


def _sc_gather(x, sorted_tok):
    rows_per_w = NA // NW
    CH = 64
    mesh = plsc.VectorSubcoreMesh(core_axis_name="c", subcore_axis_name="s")

    @functools.partial(
        pl.kernel,
        mesh=mesh,
        out_type=jax.ShapeDtypeStruct((NA, DM), jnp.float32),
        scratch_types=[
            pltpu.VMEM((CH,), jnp.int32),
            pltpu.VMEM((CH, DM), jnp.float32),
            pltpu.SemaphoreType.DMA,
        ],
    )
    def k(x_hbm, idx_hbm, out_hbm, idx_v, rows_v, sem):
        wid = lax.axis_index("s") * 2 + lax.axis_index("c")
        base = wid * rows_per_w

        def chunk(c, carry):
            o = base + c * CH
            pltpu.sync_copy(idx_hbm.at[pl.ds(o, CH)], idx_v)
            pltpu.async_copy(x_hbm.at[idx_v], rows_v, sem).wait()
            pltpu.sync_copy(rows_v, out_hbm.at[pl.ds(o, CH)])
            return carry

        lax.fori_loop(0, rows_per_w // CH, chunk, None)

    return k(x, sorted_tok)


def _sc_combine(y, pos0, pos1):
    tok_per_w = NT // NW
    C = 32
    mesh = plsc.VectorSubcoreMesh(core_axis_name="c", subcore_axis_name="s")

    @functools.partial(
        pl.kernel,
        mesh=mesh,
        out_type=jax.ShapeDtypeStruct((NT, DM), jnp.float32),
        scratch_types=[
            pltpu.VMEM((C,), jnp.int32),
            pltpu.VMEM((C,), jnp.int32),
            pltpu.VMEM((C, DM), jnp.float32),
            pltpu.VMEM((C, DM), jnp.float32),
            pltpu.VMEM((C, DM), jnp.float32),
            pltpu.SemaphoreType.DMA,
            pltpu.SemaphoreType.DMA,
        ],
    )
    def k(y_hbm, p0_hbm, p1_hbm, out_hbm, i0, i1, r0, r1, ov, s0, s1):
        wid = lax.axis_index("s") * 2 + lax.axis_index("c")
        base = wid * tok_per_w

        def chunk(c, carry):
            o = base + c * C
            pltpu.sync_copy(p0_hbm.at[pl.ds(o, C)], i0)
            pltpu.sync_copy(p1_hbm.at[pl.ds(o, C)], i1)
            cp0 = pltpu.async_copy(y_hbm.at[i0], r0, s0)
            cp1 = pltpu.async_copy(y_hbm.at[i1], r1, s1)
            cp0.wait()
            cp1.wait()

            def per_tok(t, carry2):
                def per_vec(v, carry3):
                    sl = pl.ds(v * 16, 16)
                    ov[t, sl] = r0[t, sl] + r1[t, sl]
                    return carry3

                lax.fori_loop(0, DM // 16, per_vec, None)
                return carry2

            lax.fori_loop(0, C, per_tok, None)
            pltpu.sync_copy(ov, out_hbm.at[pl.ds(o, C)])
            return carry

        lax.fori_loop(0, tok_per_w // C, chunk, None)

    return k(y, pos0, pos1)


def _ffn_body(meta_ref, xs_ref, w1_ref, w2_ref, w3_ref, sw_ref, out_ref):
    i = pl.program_id(0)
    lo = meta_ref[2, i]
    hi = meta_ref[3, i]
    first = meta_ref[4, i]

    xb = xs_ref[...].astype(jnp.bfloat16)
    w1b = w1_ref[0].astype(jnp.bfloat16)
    w2b = w2_ref[0].astype(jnp.bfloat16)
    w3b = w3_ref[0].astype(jnp.bfloat16)
    g = jax.nn.silu(jnp.dot(xb, w1b, preferred_element_type=jnp.float32))
    v = jnp.dot(xb, w2b, preferred_element_type=jnp.float32)
    h = (g * v).astype(jnp.bfloat16)
    o = jnp.dot(h, w3b, preferred_element_type=jnp.float32)

    ridx = lax.broadcasted_iota(jnp.int32, (BM, 1), 0)
    keep = (ridx >= lo) & (ridx < hi)
    o = jnp.where(keep, o * sw_ref[...], 0.0)

    @pl.when(first == 1)
    def _():
        out_ref[...] = o

    @pl.when(first == 0)
    def _():
        out_ref[...] += o


def _tc_ffn(xs, w1, w2, w3, sw, meta):
    grid_spec = pltpu.PrefetchScalarGridSpec(
        num_scalar_prefetch=1,
        grid=(TMAX,),
        in_specs=[
            pl.BlockSpec((BM, DM), lambda i, m: (m[0, i], 0)),
            pl.BlockSpec((1, DM, DF), lambda i, m: (m[1, i], 0, 0)),
            pl.BlockSpec((1, DM, DF), lambda i, m: (m[1, i], 0, 0)),
            pl.BlockSpec((1, DF, DM), lambda i, m: (m[1, i], 0, 0)),
            pl.BlockSpec((BM, 1), lambda i, m: (m[0, i], 0)),
        ],
        out_specs=pl.BlockSpec((BM, DM), lambda i, m: (m[0, i], 0)),
    )
    return pl.pallas_call(
        _ffn_body,
        grid_spec=grid_spec,
        out_shape=jax.ShapeDtypeStruct((NA, DM), jnp.float32),
        compiler_params=pltpu.CompilerParams(
            dimension_semantics=("arbitrary",),
        ),
    )(meta, xs, w1, w2, w3, sw)


def kernel(x, expert_indices, expert_weights, w1, w2, w3):
    flat_e = expert_indices.reshape(-1)
    iota = jnp.arange(NA, dtype=jnp.int32)
    sorted_e, order = lax.sort_key_val(flat_e, iota)
    sorted_tok = (order // TK).astype(jnp.int32)
    _, inv = lax.sort_key_val(order, iota)
    eids = jnp.arange(NE, dtype=flat_e.dtype)
    starts = jnp.searchsorted(sorted_e, eids, side="left").astype(jnp.int32)
    ends = jnp.searchsorted(sorted_e, eids, side="right").astype(jnp.int32)

    blo = (jnp.arange(NB, dtype=jnp.int32) * BM)[:, None]
    s = starts[None, :].astype(jnp.int32)
    en = ends[None, :].astype(jnp.int32)
    hit = (s < blo + BM) & (en > blo)
    flat_hit = hit.reshape(-1)
    tile_idx = jnp.cumsum(flat_hit) - 1
    target = jnp.where(flat_hit, tile_idx, TMAX).astype(jnp.int32)

    bb = jnp.broadcast_to(jnp.arange(NB, dtype=jnp.int32)[:, None], (NB, NE))
    ee = jnp.broadcast_to(jnp.arange(NE, dtype=jnp.int32)[None, :], (NB, NE))
    lo = jnp.maximum(s - blo, 0).astype(jnp.int32)
    hi = jnp.minimum(en - blo, BM).astype(jnp.int32)

    block_a = jnp.full((TMAX,), NB - 1, jnp.int32).at[target].set(
        bb.reshape(-1), mode="drop")
    exp_a = jnp.full((TMAX,), NE - 1, jnp.int32).at[target].set(
        ee.reshape(-1), mode="drop")
    lo_a = jnp.zeros((TMAX,), jnp.int32).at[target].set(
        lo.reshape(-1), mode="drop")
    hi_a = jnp.zeros((TMAX,), jnp.int32).at[target].set(
        hi.reshape(-1), mode="drop")
    first_a = jnp.concatenate(
        [jnp.ones((1,), jnp.int32),
         (block_a[1:] != block_a[:-1]).astype(jnp.int32)])
    meta = jnp.stack([block_a, exp_a, lo_a, hi_a, first_a])

    sw = expert_weights.reshape(-1)[order].reshape(NA, 1)

    xs = _sc_gather(x, sorted_tok)
    y = _tc_ffn(xs, w1, w2, w3, sw, meta)

    pos = inv.reshape(NT, TK)
    out = _sc_combine(y, pos[:, 0], pos[:, 1])
    return out
